# Initial kernel scaffold; baseline (speedup 1.0000x reference)
import jax
import jax.numpy as jnp
from jax.experimental import pallas as pl


def kernel(features, geometry, W1, W2):
    raise NotImplementedError("write your pallas kernel here")


# speedup vs baseline: 70.9906x; 70.9906x over previous
import jax
import jax.numpy as jnp
from jax.experimental import pallas as pl
from jax.experimental.pallas import tpu as pltpu

RADIUS = 0.2
C_IN = 32
C_OUT = 32
HIDDEN = 64

A_BLK = 128
B_BLK = 128


def _g_matmul(f_ref, w_ref, out_ref):
    out_ref[...] = jnp.dot(
        f_ref[...], w_ref[...],
        preferred_element_type=jnp.float32,
        precision=jax.lax.Precision.HIGHEST,
    )


def _conv_kernel(ga_ref, gbT_ref, w1_ref, w1T_ref, g_ref, out_ref, hm_ref):
    bo = pl.program_id(2)
    ga = ga_ref[0]
    gbT = gbT_ref[0]
    pa = jnp.dot(ga, w1_ref[...], preferred_element_type=jnp.float32,
                 precision=jax.lax.Precision.HIGHEST)
    pbT = jnp.dot(w1T_ref[...], gbT, preferred_element_type=jnp.float32,
                  precision=jax.lax.Precision.HIGHEST)
    d0 = gbT[0:1, :] - ga[:, 0:1]
    d1 = gbT[1:2, :] - ga[:, 1:2]
    d2 = gbT[2:3, :] - ga[:, 2:3]
    n2 = d0 * d0 + d1 * d1 + d2 * d2
    m = (jnp.sqrt(n2) < RADIUS).astype(jnp.float32)
    for k in range(HIDDEN):
        hk = jnp.maximum(pbT[k:k + 1, :] - pa[:, k:k + 1], 0.0) * m
        hm_ref[:, k * B_BLK:(k + 1) * B_BLK] = hk
    acc = jnp.dot(
        hm_ref[...], g_ref[0, 0],
        preferred_element_type=jnp.float32,
        precision=jax.lax.Precision.HIGHEST,
    )

    @pl.when(bo == 0)
    def _():
        out_ref[0] = acc

    @pl.when(bo != 0)
    def _():
        out_ref[0] = out_ref[0] + acc


def kernel(features, geometry, W1, W2):
    batch, n, _ = geometry.shape
    n_ao = n // A_BLK
    n_bo = n // B_BLK

    w2t = W2.reshape(HIDDEN, C_OUT, C_IN).transpose(2, 0, 1).reshape(C_IN, HIDDEN * C_OUT)
    g1 = pl.pallas_call(
        _g_matmul,
        out_shape=jax.ShapeDtypeStruct((batch * n, HIDDEN * C_OUT), jnp.float32),
    )(features.reshape(batch * n, C_IN), w2t)
    g = (
        g1.reshape(batch, n_bo, B_BLK, HIDDEN, C_OUT)
        .transpose(0, 1, 3, 2, 4)
        .reshape(batch, n_bo, HIDDEN * B_BLK, C_OUT)
    )

    gT = geometry.transpose(0, 2, 1)
    w1T = W1.T

    out = pl.pallas_call(
        _conv_kernel,
        grid=(batch, n_ao, n_bo),
        in_specs=[
            pl.BlockSpec((1, A_BLK, 3), lambda z, ao, bo: (z, ao, 0)),
            pl.BlockSpec((1, 3, B_BLK), lambda z, ao, bo: (z, 0, bo)),
            pl.BlockSpec((3, HIDDEN), lambda z, ao, bo: (0, 0)),
            pl.BlockSpec((HIDDEN, 3), lambda z, ao, bo: (0, 0)),
            pl.BlockSpec((1, 1, HIDDEN * B_BLK, C_OUT), lambda z, ao, bo: (z, bo, 0, 0)),
        ],
        out_specs=pl.BlockSpec((1, A_BLK, C_OUT), lambda z, ao, bo: (z, ao, 0)),
        out_shape=jax.ShapeDtypeStruct((batch, n, C_OUT), jnp.float32),
        scratch_shapes=[pltpu.VMEM((A_BLK, HIDDEN * B_BLK), jnp.float32)],
    )(geometry, gT, W1, w1T, g)
    return out


# speedup vs baseline: 135.2446x; 1.9051x over previous
import jax
import jax.numpy as jnp
from jax.experimental import pallas as pl
from jax.experimental.pallas import tpu as pltpu

RADIUS = 0.2
C_IN = 32
C_OUT = 32
HIDDEN = 64

A_BLK = 256
B_BLK = 128


def _g_matmul(f_ref, w_ref, out_ref):
    out_ref[...] = jnp.dot(
        f_ref[...], w_ref[...],
        preferred_element_type=jnp.float32,
        precision=jax.lax.Precision.HIGHEST,
    )


def _conv_kernel(ga_ref, gbT_ref, w1_ref, w1T_ref, g_ref, out_ref, hm_ref):
    bo = pl.program_id(2)
    ga = ga_ref[0]
    gbT = gbT_ref[0]
    pa = jnp.dot(ga, w1_ref[...], preferred_element_type=jnp.float32,
                 precision=jax.lax.Precision.HIGHEST)
    pbT = jnp.dot(w1T_ref[...], gbT, preferred_element_type=jnp.float32,
                  precision=jax.lax.Precision.HIGHEST)
    d0 = gbT[0:1, :] - ga[:, 0:1]
    d1 = gbT[1:2, :] - ga[:, 1:2]
    d2 = gbT[2:3, :] - ga[:, 2:3]
    n2 = d0 * d0 + d1 * d1 + d2 * d2
    m = (jnp.sqrt(n2) < RADIUS).astype(jnp.float32)
    for k in range(HIDDEN):
        hk = jnp.maximum(pbT[k:k + 1, :] - pa[:, k:k + 1], 0.0) * m
        hm_ref[:, k * B_BLK:(k + 1) * B_BLK] = hk
    acc = jnp.dot(
        hm_ref[...], g_ref[0, 0],
        preferred_element_type=jnp.float32,
        precision=jax.lax.Precision.DEFAULT,
    )

    @pl.when(bo == 0)
    def _():
        out_ref[0] = acc

    @pl.when(bo != 0)
    def _():
        out_ref[0] = out_ref[0] + acc


def kernel(features, geometry, W1, W2):
    batch, n, _ = geometry.shape
    n_ao = n // A_BLK
    n_bo = n // B_BLK

    w2t = W2.reshape(HIDDEN, C_OUT, C_IN).transpose(2, 0, 1).reshape(C_IN, HIDDEN * C_OUT)
    g1 = pl.pallas_call(
        _g_matmul,
        out_shape=jax.ShapeDtypeStruct((batch * n, HIDDEN * C_OUT), jnp.float32),
    )(features.reshape(batch * n, C_IN), w2t)
    g = (
        g1.reshape(batch, n_bo, B_BLK, HIDDEN, C_OUT)
        .transpose(0, 1, 3, 2, 4)
        .reshape(batch, n_bo, HIDDEN * B_BLK, C_OUT)
    )

    gT = geometry.transpose(0, 2, 1)
    w1T = W1.T

    out = pl.pallas_call(
        _conv_kernel,
        grid=(batch, n_ao, n_bo),
        in_specs=[
            pl.BlockSpec((1, A_BLK, 3), lambda z, ao, bo: (z, ao, 0)),
            pl.BlockSpec((1, 3, B_BLK), lambda z, ao, bo: (z, 0, bo)),
            pl.BlockSpec((3, HIDDEN), lambda z, ao, bo: (0, 0)),
            pl.BlockSpec((HIDDEN, 3), lambda z, ao, bo: (0, 0)),
            pl.BlockSpec((1, 1, HIDDEN * B_BLK, C_OUT), lambda z, ao, bo: (z, bo, 0, 0)),
        ],
        out_specs=pl.BlockSpec((1, A_BLK, C_OUT), lambda z, ao, bo: (z, ao, 0)),
        out_shape=jax.ShapeDtypeStruct((batch, n, C_OUT), jnp.float32),
        scratch_shapes=[pltpu.VMEM((A_BLK, HIDDEN * B_BLK), jnp.float32)],
    )(geometry, gT, W1, w1T, g)
    return out
